# baseline (speedup 1.0000x reference)
import jax
import jax.numpy as jnp
from jax import lax
from jax.experimental import pallas as pl

_THRESHOLD = 0.5
_VAR0, _VAR1 = 0.1, 0.2
_C = 21
_NEGPOS = 3
_O = 16
_P = 24564
_R1, _L1 = 12, 2047
_R2, _L2 = 4094, 126


def _match_kernel(priors_ref, localT_ref, bboxes_ref, labels_ref,
                  conf_t_ref, loss_l_ref, npos_b_ref, npos_tot_ref):
    @pl.when(pl.program_id(0) == 0)
    def _init():
        loss_l_ref[...] = jnp.zeros((1, 1), jnp.float32)
        npos_tot_ref[...] = jnp.zeros((1, 1), jnp.float32)

    pr = priors_ref[...]
    p_cx, p_cy, p_w, p_h = pr[0], pr[1], pr[2], pr[3]
    px1 = p_cx - p_w * 0.5
    py1 = p_cy - p_h * 0.5
    px2 = p_cx + p_w * 0.5
    py2 = p_cy + p_h * 0.5
    area_p = (px2 - px1) * (py2 - py1)

    truths = bboxes_ref[0]
    labels = labels_ref[0, 0]

    pid = (_L1 * lax.broadcasted_iota(jnp.int32, (_R1, _L1), 0)
           + lax.broadcasted_iota(jnp.int32, (_R1, _L1), 1))

    bto = jnp.full((_R1, _L1), -1.0, jnp.float32)
    bti = jnp.zeros((_R1, _L1), jnp.int32)
    bpi = []
    for o in range(_O):
        tx1, ty1, tx2, ty2 = truths[o, 0], truths[o, 1], truths[o, 2], truths[o, 3]
        iw = jnp.maximum(jnp.minimum(tx2, px2) - jnp.maximum(tx1, px1), 0.0)
        ih = jnp.maximum(jnp.minimum(ty2, py2) - jnp.maximum(ty1, py1), 0.0)
        inter = iw * ih
        area_t = (tx2 - tx1) * (ty2 - ty1)
        ov = inter / (area_t + area_p - inter)
        better = ov > bto
        bti = jnp.where(better, o, bti)
        bto = jnp.maximum(bto, ov)
        rmax = jnp.max(ov)
        bpi.append(jnp.min(jnp.where(ov == rmax, pid, _P)))

    forced = jnp.full((_R1, _L1), -1, jnp.int32)
    for o in range(_O):
        forced = jnp.where(pid == bpi[o], o, forced)
    is_forced = forced >= 0
    bti = jnp.where(is_forced, forced, bti)
    bto = jnp.where(is_forced, 2.0, bto)

    mx1 = jnp.zeros((_R1, _L1), jnp.float32)
    my1 = jnp.zeros((_R1, _L1), jnp.float32)
    mx2 = jnp.zeros((_R1, _L1), jnp.float32)
    my2 = jnp.zeros((_R1, _L1), jnp.float32)
    mlab = jnp.zeros((_R1, _L1), jnp.int32)
    for o in range(_O):
        sel = bti == o
        mx1 = jnp.where(sel, truths[o, 0], mx1)
        my1 = jnp.where(sel, truths[o, 1], my1)
        mx2 = jnp.where(sel, truths[o, 2], mx2)
        my2 = jnp.where(sel, truths[o, 3], my2)
        mlab = jnp.where(sel, labels[o], mlab)

    pos = bto >= _THRESHOLD
    posf = pos.astype(jnp.float32)
    conf_t = jnp.where(pos, mlab + 1, 0)
    conf_t_ref[0] = conf_t

    g_cx = ((mx1 + mx2) * 0.5 - p_cx) / (_VAR0 * p_w)
    g_cy = ((my1 + my2) * 0.5 - p_cy) / (_VAR0 * p_h)
    g_w = jnp.log((mx2 - mx1) / p_w) / _VAR1
    g_h = jnp.log((my2 - my1) / p_h) / _VAR1

    lx = localT_ref[0]

    def _sl1(d):
        ad = jnp.abs(d)
        return jnp.where(ad < 1.0, 0.5 * d * d, ad - 0.5)

    sl1 = (_sl1(lx[0] - g_cx) + _sl1(lx[1] - g_cy)
           + _sl1(lx[2] - g_w) + _sl1(lx[3] - g_h))
    loss_l_part = jnp.sum(sl1 * posf)

    npos = jnp.sum(posf)
    npos_b_ref[...] = jnp.full((1, 1, 128), jnp.sum(pos.astype(jnp.int32)),
                               jnp.int32)
    loss_l_ref[...] += loss_l_part.reshape(1, 1)
    npos_tot_ref[...] += npos.reshape(1, 1)


def _ce_kernel(conf_ref, ct6_ref, v6_ref, cepos_ref):
    @pl.when(pl.program_id(0) == 0)
    def _init():
        cepos_ref[...] = jnp.zeros((1, 1), jnp.float32)

    X = conf_ref[0]
    ct = ct6_ref[0]

    m = jnp.max(X)
    E = jnp.exp(X - m)

    a2 = lax.broadcasted_iota(jnp.int32, (_L2, _L2), 0) // _C
    b2 = lax.broadcasted_iota(jnp.int32, (_L2, _L2), 1) // _C
    M2 = (a2 == b2).astype(jnp.float32)
    S = lax.dot_general(E, M2, (((1,), (0,)), ((), ())),
                        preferred_element_type=jnp.float32)
    lse = jnp.log(S) + m

    ru = lax.broadcasted_iota(jnp.int32, (6, _L2), 0)
    rm = lax.broadcasted_iota(jnp.int32, (6, _L2), 1) // _C
    R = (ru == rm).astype(jnp.float32)
    ct_rep = lax.dot_general(ct.astype(jnp.float32), R,
                             (((1,), (0,)), ((), ())),
                             preferred_element_type=jnp.float32)

    cmod = (lax.broadcasted_iota(jnp.int32, (1, _L2), 1) % _C).astype(jnp.float32)
    onehot = (cmod == ct_rep).astype(jnp.float32)
    diffci = lse - X
    ispos = (ct_rep >= 0.5).astype(jnp.float32)
    cepos_part = jnp.sum(diffci * onehot * ispos)
    v_rep = diffci * onehot * (1.0 - ispos)

    cu = lax.broadcasted_iota(jnp.int32, (_L2, 6), 0) // _C
    cv = lax.broadcasted_iota(jnp.int32, (_L2, 6), 1)
    Mc = (cu == cv).astype(jnp.float32)
    v6 = lax.dot_general(v_rep, Mc, (((1,), (0,)), ((), ())),
                         preferred_element_type=jnp.float32)
    v6_ref[0] = v6
    cepos_ref[...] += cepos_part.reshape(1, 1)


def _select_kernel(v_ref, npos_ref, negsum_ref):
    @pl.when(pl.program_id(0) == 0)
    def _init():
        negsum_ref[...] = jnp.zeros((1, 1), jnp.float32)

    v = jnp.maximum(v_ref[...], 0.0)
    vb = lax.bitcast_convert_type(v, jnp.int32)
    k = jnp.minimum(_NEGPOS * npos_ref[:, 0, 0:1], _P - 1)

    def body(i, prefix):
        cand = prefix | jnp.left_shift(jnp.int32(1), 30 - i)
        cnt = jnp.sum((vb >= cand).astype(jnp.int32), axis=1, keepdims=True)
        return jnp.where(cnt >= k, cand, prefix)

    t = lax.fori_loop(0, 31, body, jnp.zeros((8, 1), jnp.int32))
    gt = vb > t
    cnt_gt = jnp.sum(gt.astype(jnp.int32), axis=1, keepdims=True)
    tf = lax.bitcast_convert_type(t, jnp.float32)
    part = (jnp.sum(jnp.where(gt, v, 0.0))
            + jnp.sum((k - cnt_gt).astype(jnp.float32) * tf))
    negsum_ref[...] += part.reshape(1, 1)


@jax.jit
def kernel(local, conf, priors, bboxes, labels):
    B = local.shape[0]
    localT = jnp.transpose(local, (0, 2, 1)).reshape(B, 4, _R1, _L1)
    priorsT = jnp.transpose(priors, (1, 0)).reshape(4, _R1, _L1)
    conf3 = conf.reshape(B, _R2, _L2)
    labels3 = labels.astype(jnp.int32).reshape(B, 1, _O)

    conf_t, loss_l, npos_b, npos_tot = pl.pallas_call(
        _match_kernel,
        grid=(B,),
        in_specs=[
            pl.BlockSpec((4, _R1, _L1), lambda b: (0, 0, 0)),
            pl.BlockSpec((1, 4, _R1, _L1), lambda b: (b, 0, 0, 0)),
            pl.BlockSpec((1, _O, 4), lambda b: (b, 0, 0)),
            pl.BlockSpec((1, 1, _O), lambda b: (b, 0, 0)),
        ],
        out_specs=[
            pl.BlockSpec((1, _R1, _L1), lambda b: (b, 0, 0)),
            pl.BlockSpec((1, 1), lambda b: (0, 0)),
            pl.BlockSpec((1, 1, 128), lambda b: (b, 0, 0)),
            pl.BlockSpec((1, 1), lambda b: (0, 0)),
        ],
        out_shape=[
            jax.ShapeDtypeStruct((B, _R1, _L1), jnp.int32),
            jax.ShapeDtypeStruct((1, 1), jnp.float32),
            jax.ShapeDtypeStruct((B, 1, 128), jnp.int32),
            jax.ShapeDtypeStruct((1, 1), jnp.float32),
        ],
    )(priorsT, localT, bboxes, labels3)

    ct6 = conf_t.reshape(B, _R2, 6)

    v6, cepos = pl.pallas_call(
        _ce_kernel,
        grid=(B,),
        in_specs=[
            pl.BlockSpec((1, _R2, _L2), lambda b: (b, 0, 0)),
            pl.BlockSpec((1, _R2, 6), lambda b: (b, 0, 0)),
        ],
        out_specs=[
            pl.BlockSpec((1, _R2, 6), lambda b: (b, 0, 0)),
            pl.BlockSpec((1, 1), lambda b: (0, 0)),
        ],
        out_shape=[
            jax.ShapeDtypeStruct((B, _R2, 6), jnp.float32),
            jax.ShapeDtypeStruct((1, 1), jnp.float32),
        ],
    )(conf3, ct6)

    v = v6.reshape(B, _P)

    negsum = pl.pallas_call(
        _select_kernel,
        grid=(B // 8,),
        in_specs=[
            pl.BlockSpec((8, _P), lambda g: (g, 0)),
            pl.BlockSpec((8, 1, 128), lambda g: (g, 0, 0)),
        ],
        out_specs=pl.BlockSpec((1, 1), lambda g: (0, 0)),
        out_shape=jax.ShapeDtypeStruct((1, 1), jnp.float32),
    )(v, npos_b)

    n = npos_tot[0, 0]
    return (loss_l[0, 0] / n, (cepos[0, 0] + negsum[0, 0]) / n)
